# baseline (device time: 2461754 ns/iter reference)
import jax
import jax.numpy as jnp
from jax import lax
from jax.experimental import pallas as pl
from jax.experimental.pallas import tpu as pltpu

B = 32
NB = 256
BS = 32
H = 16
D = 128
PAGES_LOCAL = 256
SCALE = D ** -0.5


def kernel(Q, K, V, bt, lens):
    my_x = lax.axis_index("x")
    off = my_x * PAGES_LOCAL
    btl = bt - off
    j_iota = jnp.arange(NB, dtype=jnp.int32)[None, :]
    valid = (btl >= 0) & (btl < PAGES_LOCAL) & (j_iota < lens[:, None])
    p_masked = jnp.where(valid, btl, 0).astype(jnp.int32)
    lastj = lax.cummax(jnp.where(valid, j_iota, -1), axis=1)
    lastj = jnp.maximum(lastj, 0)
    p_ff = jnp.take_along_axis(p_masked, lastj, axis=1)
    valid_i32 = valid.astype(jnp.int32)

    def body(pff_ref, valid_ref, q_ref, k_ref, v_ref, out_ref,
             acc_ref, l_ref, pacc_ref, pl_ref, send_sems, recv_sems):
        i = pl.program_id(0)
        j = pl.program_id(1)
        mx = lax.axis_index("x")
        partner = (1 - mx, lax.axis_index("y"), lax.axis_index("z"))

        barrier_sem = pltpu.get_barrier_semaphore()

        @pl.when((i == 0) & (j == 0))
        def _():
            pl.semaphore_signal(
                barrier_sem, inc=1,
                device_id=partner, device_id_type=pl.DeviceIdType.MESH,
            )
            pl.semaphore_wait(barrier_sem, 1)

        @pl.when(j == 0)
        def _():
            acc_ref[pl.ds(i, 1)] = jnp.zeros((1, H, D), jnp.float32)
            l_ref[pl.ds(i, 1)] = jnp.zeros((1, H), jnp.float32)

        @pl.when(valid_ref[i, j] == 1)
        def _():
            q = q_ref[0, 0].astype(jnp.float32)
            k = k_ref[0].astype(jnp.float32)
            v = v_ref[0].astype(jnp.float32)
            s = jnp.sum(k * q[None], axis=-1) * SCALE
            p = jnp.exp(s)
            l_ref[pl.ds(i, 1)] = l_ref[pl.ds(i, 1)] + jnp.sum(
                p, axis=0, keepdims=True)
            pv = jnp.sum(v * p[:, :, None], axis=0)
            acc_ref[pl.ds(i, 1)] = acc_ref[pl.ds(i, 1)] + pv[None]

        @pl.when((i == B - 1) & (j == NB - 1))
        def _():
            rdma_acc = pltpu.make_async_remote_copy(
                src_ref=acc_ref, dst_ref=pacc_ref,
                send_sem=send_sems.at[0], recv_sem=recv_sems.at[0],
                device_id=partner, device_id_type=pl.DeviceIdType.MESH,
            )
            rdma_l = pltpu.make_async_remote_copy(
                src_ref=l_ref, dst_ref=pl_ref,
                send_sem=send_sems.at[1], recv_sem=recv_sems.at[1],
                device_id=partner, device_id_type=pl.DeviceIdType.MESH,
            )
            rdma_acc.start()
            rdma_l.start()
            rdma_acc.wait()
            rdma_l.wait()
            denom = l_ref[...] + pl_ref[...]
            merged = (acc_ref[...] + pacc_ref[...]) / denom[:, :, None]
            out_ref[:, 0, :, :] = merged

    grid_spec = pltpu.PrefetchScalarGridSpec(
        num_scalar_prefetch=2,
        grid=(B, NB),
        in_specs=[
            pl.BlockSpec((1, 1, H, D), lambda i, j, pff, val: (i, 0, 0, 0)),
            pl.BlockSpec((1, BS, H, D),
                         lambda i, j, pff, val: (pff[i, j], 0, 0, 0)),
            pl.BlockSpec((1, BS, H, D),
                         lambda i, j, pff, val: (pff[i, j], 0, 0, 0)),
        ],
        out_specs=pl.BlockSpec((B, 1, H, D),
                               lambda i, j, pff, val: (0, 0, 0, 0)),
        scratch_shapes=[
            pltpu.VMEM((B, H, D), jnp.float32),
            pltpu.VMEM((B, H), jnp.float32),
            pltpu.VMEM((B, H, D), jnp.float32),
            pltpu.VMEM((B, H), jnp.float32),
            pltpu.SemaphoreType.DMA((2,)),
            pltpu.SemaphoreType.DMA((2,)),
        ],
    )

    return pl.pallas_call(
        body,
        grid_spec=grid_spec,
        out_shape=jax.ShapeDtypeStruct((B, 1, H, D), jnp.float32),
        compiler_params=pltpu.CompilerParams(
            dimension_semantics=("arbitrary", "arbitrary"),
            collective_id=0,
        ),
    )(p_ff, valid_i32, Q, K, V)


# device time: 551776 ns/iter; 4.4615x vs baseline; 4.4615x over previous
import jax
import jax.numpy as jnp
from jax import lax
from jax.experimental import pallas as pl
from jax.experimental.pallas import tpu as pltpu

B = 32
NB = 256
BS = 32
H = 16
D = 128
PAGES_LOCAL = 256
C = 8
CK = C * BS
SCALE = D ** -0.5


def kernel(Q, K, V, bt, lens):
    my_x = lax.axis_index("x")
    off = my_x * PAGES_LOCAL
    btl = bt - off
    j_iota = jnp.arange(NB, dtype=jnp.int32)[None, :]
    valid = (btl >= 0) & (btl < PAGES_LOCAL) & (j_iota < lens[:, None])
    order = jnp.argsort(jnp.logical_not(valid), axis=1, stable=True)
    loc = jnp.take_along_axis(
        jnp.where(valid, btl, 0).astype(jnp.int32), order, axis=1)
    nloc = valid.sum(axis=1).astype(jnp.int32)

    def body(loc_ref, nloc_ref, q_ref, k_hbm, v_hbm, out_ref,
             kbuf, vbuf, acc_ref, l_ref, pacc_ref, pl_ref,
             copy_sems, send_sems, recv_sems):
        i = pl.program_id(0)
        partner = (1 - lax.axis_index("x"), lax.axis_index("y"),
                   lax.axis_index("z"))

        barrier_sem = pltpu.get_barrier_semaphore()

        @pl.when(i == 0)
        def _():
            pl.semaphore_signal(
                barrier_sem, inc=1,
                device_id=partner, device_id_type=pl.DeviceIdType.MESH,
            )
            pl.semaphore_wait(barrier_sem, 1)

        n = nloc_ref[i]
        nchunks = (n + C - 1) // C

        def start_chunk(c, slot):
            for s in range(C):
                p = loc_ref[i, c * C + s]
                pltpu.make_async_copy(
                    k_hbm.at[p], kbuf.at[slot, pl.ds(s * BS, BS)],
                    copy_sems.at[slot]).start()
                pltpu.make_async_copy(
                    v_hbm.at[p], vbuf.at[slot, pl.ds(s * BS, BS)],
                    copy_sems.at[slot]).start()

        def wait_chunk(slot):
            for s in range(C):
                pltpu.make_async_copy(
                    k_hbm.at[0], kbuf.at[slot, pl.ds(s * BS, BS)],
                    copy_sems.at[slot]).wait()
                pltpu.make_async_copy(
                    v_hbm.at[0], vbuf.at[slot, pl.ds(s * BS, BS)],
                    copy_sems.at[slot]).wait()

        @pl.when(nchunks > 0)
        def _():
            start_chunk(0, 0)

        q = q_ref[0, 0]

        def loop_body(c, carry):
            acc, l = carry
            slot = lax.rem(c, 2)

            @pl.when(c + 1 < nchunks)
            def _():
                start_chunk(c + 1, 1 - slot)

            wait_chunk(slot)
            kc = kbuf[slot]
            vc = vbuf[slot]
            s = jnp.sum(kc * q[None], axis=-1) * SCALE
            kidx = lax.broadcasted_iota(jnp.int32, (CK, 1), 0)
            in_range = (kidx + c * CK) < n * BS
            p = jnp.where(in_range, jnp.exp(s), 0.0)
            l = l + jnp.sum(p, axis=0, keepdims=True)
            acc = acc + jnp.sum(vc * p[:, :, None], axis=0)
            return acc, l

        acc, l = lax.fori_loop(
            0, nchunks, loop_body,
            (jnp.zeros((H, D), jnp.float32), jnp.zeros((1, H), jnp.float32)),
        )
        acc_ref[pl.ds(i, 1)] = acc[None]
        l_ref[pl.ds(i, 1)] = l

        @pl.when(i == B - 1)
        def _():
            rdma_acc = pltpu.make_async_remote_copy(
                src_ref=acc_ref, dst_ref=pacc_ref,
                send_sem=send_sems.at[0], recv_sem=recv_sems.at[0],
                device_id=partner, device_id_type=pl.DeviceIdType.MESH,
            )
            rdma_l = pltpu.make_async_remote_copy(
                src_ref=l_ref, dst_ref=pl_ref,
                send_sem=send_sems.at[1], recv_sem=recv_sems.at[1],
                device_id=partner, device_id_type=pl.DeviceIdType.MESH,
            )
            rdma_acc.start()
            rdma_l.start()
            rdma_acc.wait()
            rdma_l.wait()
            denom = l_ref[...] + pl_ref[...]
            merged = (acc_ref[...] + pacc_ref[...]) / denom[:, :, None]
            out_ref[:, 0, :, :] = merged

    grid_spec = pltpu.PrefetchScalarGridSpec(
        num_scalar_prefetch=2,
        grid=(B,),
        in_specs=[
            pl.BlockSpec((1, 1, H, D), lambda i, locp, nl: (i, 0, 0, 0)),
            pl.BlockSpec(memory_space=pl.ANY),
            pl.BlockSpec(memory_space=pl.ANY),
        ],
        out_specs=pl.BlockSpec((B, 1, H, D), lambda i, locp, nl: (0, 0, 0, 0)),
        scratch_shapes=[
            pltpu.VMEM((2, CK, H, D), jnp.float32),
            pltpu.VMEM((2, CK, H, D), jnp.float32),
            pltpu.VMEM((B, H, D), jnp.float32),
            pltpu.VMEM((B, H), jnp.float32),
            pltpu.VMEM((B, H, D), jnp.float32),
            pltpu.VMEM((B, H), jnp.float32),
            pltpu.SemaphoreType.DMA((2,)),
            pltpu.SemaphoreType.DMA((2,)),
            pltpu.SemaphoreType.DMA((2,)),
        ],
    )

    return pl.pallas_call(
        body,
        grid_spec=grid_spec,
        out_shape=jax.ShapeDtypeStruct((B, 1, H, D), jnp.float32),
        compiler_params=pltpu.CompilerParams(
            dimension_semantics=("arbitrary",),
            collective_id=0,
        ),
    )(loc, nloc, Q, K, V)
